# baseline (speedup 1.0000x reference)
import jax
import jax.numpy as jnp
from jax.experimental import pallas as pl
from jax.experimental.pallas import tpu as pltpu
from jax.experimental.pallas import tpu_sc as plsc

_VOCAB = 100000
_EMBED = 128
_BATCH = 1024

_GATHER_WINDOW = 128

_BN = 2048
_NB = pl.cdiv(_VOCAB, _BN)
_BN_LAST = _VOCAB - (_NB - 1) * _BN
_BN_LAST_ALIGNED = (_BN_LAST // 128) * 128
_TAIL = _BN_LAST - _BN_LAST_ALIGNED
_SLOTS = 4


def _sc_gather(emb_table, indices):
    mesh = plsc.VectorSubcoreMesh(core_axis_name="core",
                                  subcore_axis_name="subcore")
    idx2d = indices.reshape(1, _BATCH)

    @pl.kernel(
        out_type=jax.ShapeDtypeStruct((_BATCH, _EMBED), emb_table.dtype),
        mesh=mesh,
    )
    def gather_kernel(tbl_hbm, idx_hbm, out_hbm):
        def body(idx_vmem, out_vmem):
            pltpu.sync_copy(tbl_hbm.at[idx_vmem.at[0]], out_vmem)

        pltpu.emit_pipeline(
            body,
            grid=(_BATCH // _GATHER_WINDOW,),
            in_specs=[pl.BlockSpec((1, _GATHER_WINDOW),
                                   index_map=lambda i: (0, i))],
            out_specs=[pl.BlockSpec((_GATHER_WINDOW, _EMBED),
                                    index_map=lambda i: (i, 0))],
            core_axis_name=("core", "subcore"),
            dimension_semantics=(pltpu.PARALLEL,),
        )(idx_hbm, out_hbm)

    return gather_kernel(emb_table, idx2d)


def _proj_body(h_ref, w_ref, b_ref, o_hbm, tail_ref, scratch, sems):
    j = pl.program_id(0)
    slot = jax.lax.rem(j, _SLOTS)

    @pl.when(j >= _SLOTS)
    def _():
        pltpu.make_async_copy(
            scratch.at[slot],
            o_hbm.at[:, pl.ds((j - _SLOTS) * _BN, _BN)],
            sems.at[slot],
        ).wait()

    h = h_ref[...].astype(jnp.bfloat16)
    w = w_ref[...].astype(jnp.bfloat16)
    acc = jax.lax.dot_general(
        h, w,
        dimension_numbers=(((1,), (1,)), ((), ())),
        preferred_element_type=jnp.float32,
    )
    acc = acc + b_ref[...]
    scratch[slot] = acc

    @pl.when(j < _NB - 1)
    def _():
        pltpu.make_async_copy(
            scratch.at[slot],
            o_hbm.at[:, pl.ds(j * _BN, _BN)],
            sems.at[slot],
        ).start()

    @pl.when(j == _NB - 1)
    def _():
        pltpu.make_async_copy(
            scratch.at[slot, :, pl.ds(0, _BN_LAST_ALIGNED)],
            o_hbm.at[:, pl.ds((_NB - 1) * _BN, _BN_LAST_ALIGNED)],
            sems.at[slot],
        ).start()
        tail_ref[...] = acc[:, _BN_LAST_ALIGNED:_BN_LAST]
        for blk in range(max(0, _NB - _SLOTS), _NB - 1):
            pltpu.make_async_copy(
                scratch.at[blk % _SLOTS],
                o_hbm.at[:, pl.ds(blk * _BN, _BN)],
                sems.at[blk % _SLOTS],
            ).wait()
        pltpu.make_async_copy(
            scratch.at[(_NB - 1) % _SLOTS, :, pl.ds(0, _BN_LAST_ALIGNED)],
            o_hbm.at[:, pl.ds((_NB - 1) * _BN, _BN_LAST_ALIGNED)],
            sems.at[(_NB - 1) % _SLOTS],
        ).wait()


def _tc_project(h, W, b2d):
    return pl.pallas_call(
        _proj_body,
        grid=(_NB,),
        in_specs=[
            pl.BlockSpec((_BATCH, _EMBED), lambda j: (0, 0)),
            pl.BlockSpec((_BN, _EMBED), lambda j: (j, 0)),
            pl.BlockSpec((1, _BN), lambda j: (0, j)),
        ],
        out_specs=[
            pl.BlockSpec(memory_space=pltpu.MemorySpace.HBM),
            pl.BlockSpec((_BATCH, _TAIL), lambda j: (0, 0)),
        ],
        out_shape=[
            jax.ShapeDtypeStruct((_BATCH, _VOCAB), jnp.float32),
            jax.ShapeDtypeStruct((_BATCH, _TAIL), jnp.float32),
        ],
        scratch_shapes=[
            pltpu.VMEM((_SLOTS, _BATCH, _BN), jnp.float32),
            pltpu.SemaphoreType.DMA((_SLOTS,)),
        ],
    )(h, W, b2d)


def _patch_body(o_in, t_ref, o_blk):
    del o_in
    o_blk[:, : _TAIL] = t_ref[...]


def _patch_tail(o, tail):
    return pl.pallas_call(
        _patch_body,
        grid=(1,),
        in_specs=[
            pl.BlockSpec(memory_space=pltpu.MemorySpace.HBM),
            pl.BlockSpec((_BATCH, _TAIL), lambda i: (0, 0)),
        ],
        out_specs=pl.BlockSpec((_BATCH, 128), lambda i: (0, _VOCAB // 128)),
        out_shape=jax.ShapeDtypeStruct((_BATCH, _VOCAB), jnp.float32),
        input_output_aliases={0: 0},
    )(o, tail)


def kernel(model_in, emb_table, W, b):
    idx = model_in.astype(jnp.int32)
    h = _sc_gather(emb_table, idx)
    o, tail = _tc_project(h, W, b.reshape(1, _VOCAB))
    return _patch_tail(o, tail)
